# baseline (speedup 1.0000x reference)
import jax
import jax.numpy as jnp
from jax.experimental import pallas as pl
from jax.experimental.pallas import tpu as pltpu

_M = 64
_K = 8
_B = 4096
_D = 2048
_BR = 1024
_NBLK = _B // _BR
_KC = 2
_DC = _D // _KC


def _router_kernel(param_ref, x_ref, t_ref, wx_ref, wt_ref,
                   gates_ref, loss_ref, acc_ref, imp_ref, load_ref):
    i = pl.program_id(0)
    k = pl.program_id(1)
    alpha = jax.nn.sigmoid(param_ref[0, 0])
    beta = 1.0 - alpha

    xb = x_ref[...].reshape(_BR, _DC)
    part = (alpha * jnp.dot(xb, wx_ref[...].reshape(_DC, _M),
                            preferred_element_type=jnp.float32) +
            beta * jnp.dot(t_ref[...], wt_ref[...],
                           preferred_element_type=jnp.float32))

    @pl.when(k == 0)
    def _first():
        acc_ref[...] = part

    @pl.when(k > 0)
    def _rest():
        acc_ref[...] += part

    @pl.when(k == _KC - 1)
    def _epilogue():
        logits = acc_ref[...]

        iota = jax.lax.broadcasted_iota(
            jnp.int32, (_BR, _M), 1).astype(jnp.float32)
        vals = logits
        mask = jnp.zeros((_BR, _M), dtype=jnp.bool_)
        neg = jnp.float32(-jnp.inf)
        big = jnp.float32(_M)
        row_max = None
        for kk in range(_K):
            m = jnp.max(vals, axis=1, keepdims=True)
            if kk == 0:
                row_max = m
            is_max = vals == m
            first = jnp.min(jnp.where(is_max, iota, big), axis=1,
                            keepdims=True)
            onehot = iota == first
            mask = jnp.logical_or(mask, onehot)
            vals = jnp.where(onehot, neg, vals)

        e = jnp.where(mask, jnp.exp(logits - row_max), 0.0)
        gates = e * (1.0 / jnp.sum(e, axis=1, keepdims=True))
        gates_ref[...] = gates

        imp_part = jnp.sum(gates, axis=0, keepdims=True)
        load_part = jnp.sum((gates > 0.0).astype(jnp.float32), axis=0,
                            keepdims=True)

        @pl.when(i == 0)
        def _init():
            imp_ref[...] = imp_part
            load_ref[...] = load_part

        @pl.when(i > 0)
        def _acc():
            imp_ref[...] += imp_part
            load_ref[...] += load_part

        @pl.when(i == _NBLK - 1)
        def _finish():
            def cv2(v):
                mean = jnp.sum(v) * (1.0 / _M)
                var = jnp.sum((v - mean) ** 2) * (1.0 / (_M - 1))
                return var / (mean * mean + 1e-10)

            total = (cv2(imp_ref[...]) + cv2(load_ref[...])) * 0.01
            loss_ref[...] = jnp.reshape(total, (1, 1))


def kernel(x, Degraded_feature, w_gate_x, w_noise_x, w_gate_text,
           w_noise_text, logit_weight_param):
    del w_noise_x, w_noise_text
    x3 = x.reshape(_B, _D // 128, 128)
    w3x = w_gate_x.reshape(_D // 128, 128, _M)
    param = jnp.asarray(logit_weight_param, jnp.float32).reshape(1, 1)
    _DCB = _DC // 128

    gates, loss = pl.pallas_call(
        _router_kernel,
        grid=(_NBLK, _KC),
        in_specs=[
            pl.BlockSpec((1, 1), lambda i, k: (0, 0)),
            pl.BlockSpec((_BR, _DCB, 128), lambda i, k: (i, k, 0)),
            pl.BlockSpec((_BR, _DC), lambda i, k: (i, k)),
            pl.BlockSpec((_DCB, 128, _M), lambda i, k: (k, 0, 0)),
            pl.BlockSpec((_DC, _M), lambda i, k: (k, 0)),
        ],
        out_specs=[
            pl.BlockSpec((_BR, _M), lambda i, k: (i, 0)),
            pl.BlockSpec((1, 1), lambda i, k: (0, 0)),
        ],
        out_shape=[
            jax.ShapeDtypeStruct((_B, _M), jnp.float32),
            jax.ShapeDtypeStruct((1, 1), jnp.float32),
        ],
        scratch_shapes=[
            pltpu.VMEM((_BR, _M), jnp.float32),
            pltpu.VMEM((1, _M), jnp.float32),
            pltpu.VMEM((1, _M), jnp.float32),
        ],
        compiler_params=pltpu.CompilerParams(
            dimension_semantics=("arbitrary", "arbitrary"),
        ),
    )(param, x3, Degraded_feature, w3x, w_gate_text)

    return (gates, loss.reshape(()))


# speedup vs baseline: 1.1799x; 1.1799x over previous
import jax
import jax.numpy as jnp
from jax.experimental import pallas as pl
from jax.experimental.pallas import tpu as pltpu

_M = 64
_K = 8
_B = 4096
_D = 2048
_BR = 1024
_NBLK = _B // _BR


def _router_kernel(param_ref, x_ref, t_ref, wx_ref, wt_ref,
                   gates_ref, loss_ref, imp_ref, load_ref):
    i = pl.program_id(0)
    alpha = jax.nn.sigmoid(param_ref[0, 0])
    beta = 1.0 - alpha

    xb = x_ref[...].reshape(_BR, _D)
    lx = jnp.dot(xb, wx_ref[...].reshape(_D, _M),
                 preferred_element_type=jnp.float32)
    lt = jnp.dot(t_ref[...], wt_ref[...], preferred_element_type=jnp.float32)
    logits = alpha * lx + beta * lt

    iota = jax.lax.broadcasted_iota(
        jnp.int32, (_BR, _M), 1).astype(jnp.float32)
    vals = logits
    mask = jnp.zeros((_BR, _M), dtype=jnp.bool_)
    neg = jnp.float32(-jnp.inf)
    big = jnp.float32(_M)
    row_max = None
    for k in range(_K):
        m = jnp.max(vals, axis=1, keepdims=True)
        if k == 0:
            row_max = m
        is_max = vals == m
        first = jnp.min(jnp.where(is_max, iota, big), axis=1, keepdims=True)
        onehot = iota == first
        mask = jnp.logical_or(mask, onehot)
        vals = jnp.where(onehot, neg, vals)

    e = jnp.where(mask, jnp.exp(logits - row_max), 0.0)
    gates = e * (1.0 / jnp.sum(e, axis=1, keepdims=True))
    gates_ref[...] = gates

    imp_part = jnp.sum(gates, axis=0, keepdims=True)
    load_part = jnp.sum((gates > 0.0).astype(jnp.float32), axis=0,
                        keepdims=True)

    @pl.when(i == 0)
    def _init():
        imp_ref[...] = imp_part
        load_ref[...] = load_part

    @pl.when(i > 0)
    def _acc():
        imp_ref[...] += imp_part
        load_ref[...] += load_part

    @pl.when(i == _NBLK - 1)
    def _finish():
        def cv2(v):
            mean = jnp.sum(v) * (1.0 / _M)
            var = jnp.sum((v - mean) ** 2) * (1.0 / (_M - 1))
            return var / (mean * mean + 1e-10)

        total = (cv2(imp_ref[...]) + cv2(load_ref[...])) * 0.01
        loss_ref[...] = jnp.reshape(total, (1, 1))


def kernel(x, Degraded_feature, w_gate_x, w_noise_x, w_gate_text,
           w_noise_text, logit_weight_param):
    del w_noise_x, w_noise_text
    x3 = x.reshape(_B, _D // 128, 128)
    w3x = w_gate_x.reshape(_D // 128, 128, _M)
    param = jnp.asarray(logit_weight_param, jnp.float32).reshape(1, 1)

    gates, loss = pl.pallas_call(
        _router_kernel,
        grid=(_NBLK,),
        in_specs=[
            pl.BlockSpec((1, 1), lambda i: (0, 0)),
            pl.BlockSpec((_BR, _D // 128, 128), lambda i: (i, 0, 0)),
            pl.BlockSpec((_BR, _D), lambda i: (i, 0)),
            pl.BlockSpec((_D // 128, 128, _M), lambda i: (0, 0, 0)),
            pl.BlockSpec((_D, _M), lambda i: (0, 0)),
        ],
        out_specs=[
            pl.BlockSpec((_BR, _M), lambda i: (i, 0)),
            pl.BlockSpec((1, 1), lambda i: (0, 0)),
        ],
        out_shape=[
            jax.ShapeDtypeStruct((_B, _M), jnp.float32),
            jax.ShapeDtypeStruct((1, 1), jnp.float32),
        ],
        scratch_shapes=[
            pltpu.VMEM((1, _M), jnp.float32),
            pltpu.VMEM((1, _M), jnp.float32),
        ],
        compiler_params=pltpu.CompilerParams(
            dimension_semantics=("arbitrary",),
        ),
    )(param, x3, Degraded_feature, w3x, w_gate_text)

    return (gates, loss.reshape(()))


# speedup vs baseline: 1.2192x; 1.0333x over previous
import jax
import jax.numpy as jnp
from jax.experimental import pallas as pl
from jax.experimental.pallas import tpu as pltpu

_M = 64
_K = 8
_B = 4096
_D = 2048
_BR = 1024
_NBLK = _B // _BR


def _router_kernel(param_ref, x_ref, t_ref, wx_ref, wt_ref,
                   gates_ref, loss_ref, imp_ref, load_ref):
    i = pl.program_id(0)
    alpha = jax.nn.sigmoid(param_ref[0, 0])
    beta = 1.0 - alpha

    xb = x_ref[...].reshape(_BR, _D)
    lx = jnp.dot(xb, wx_ref[...].reshape(_D, _M),
                 preferred_element_type=jnp.float32)
    lt = jnp.dot(t_ref[...], wt_ref[...], preferred_element_type=jnp.float32)
    logits = alpha * lx + beta * lt

    neg = jnp.float32(-jnp.inf)
    row_max = jnp.max(logits, axis=1, keepdims=True)

    def epilogue(mask):
        e = jnp.where(mask, jnp.exp(logits - row_max), 0.0)
        gates = e * (1.0 / jnp.sum(e, axis=1, keepdims=True))
        gates_ref[...] = gates

        imp_part = jnp.sum(gates, axis=0, keepdims=True)
        load_part = jnp.sum((gates > 0.0).astype(jnp.float32), axis=0,
                            keepdims=True)

        @pl.when(i == 0)
        def _init():
            imp_ref[...] = imp_part
            load_ref[...] = load_part

        @pl.when(i > 0)
        def _acc():
            imp_ref[...] += imp_part
            load_ref[...] += load_part

        @pl.when(i == _NBLK - 1)
        def _finish():
            def cv2(v):
                mean = jnp.sum(v) * (1.0 / _M)
                var = jnp.sum((v - mean) ** 2) * (1.0 / (_M - 1))
                return var / (mean * mean + 1e-10)

            total = (cv2(imp_ref[...]) + cv2(load_ref[...])) * 0.01
            loss_ref[...] = jnp.reshape(total, (1, 1))

    vals = logits
    fmask = jnp.zeros((_BR, _M), dtype=jnp.bool_)
    for k in range(_K):
        m = row_max if k == 0 else jnp.max(vals, axis=1, keepdims=True)
        sel = vals == m
        fmask = jnp.logical_or(fmask, sel)
        vals = jnp.where(sel, neg, vals)
    picked = jnp.sum(fmask.astype(jnp.float32))
    clean = picked == jnp.float32(_K * _BR)

    @pl.when(clean)
    def _fast():
        epilogue(fmask)

    @pl.when(jnp.logical_not(clean))
    def _slow():
        iota = jax.lax.broadcasted_iota(
            jnp.int32, (_BR, _M), 1).astype(jnp.float32)
        big = jnp.float32(_M)
        svals = logits
        smask = jnp.zeros((_BR, _M), dtype=jnp.bool_)
        for k in range(_K):
            m = jnp.max(svals, axis=1, keepdims=True)
            is_max = svals == m
            first = jnp.min(jnp.where(is_max, iota, big), axis=1,
                            keepdims=True)
            onehot = iota == first
            smask = jnp.logical_or(smask, onehot)
            svals = jnp.where(onehot, neg, svals)
        epilogue(smask)


def kernel(x, Degraded_feature, w_gate_x, w_noise_x, w_gate_text,
           w_noise_text, logit_weight_param):
    del w_noise_x, w_noise_text
    x3 = x.reshape(_B, _D // 128, 128)
    w3x = w_gate_x.reshape(_D // 128, 128, _M)
    param = jnp.asarray(logit_weight_param, jnp.float32).reshape(1, 1)

    gates, loss = pl.pallas_call(
        _router_kernel,
        grid=(_NBLK,),
        in_specs=[
            pl.BlockSpec((1, 1), lambda i: (0, 0)),
            pl.BlockSpec((_BR, _D // 128, 128), lambda i: (i, 0, 0)),
            pl.BlockSpec((_BR, _D), lambda i: (i, 0)),
            pl.BlockSpec((_D // 128, 128, _M), lambda i: (0, 0, 0)),
            pl.BlockSpec((_D, _M), lambda i: (0, 0)),
        ],
        out_specs=[
            pl.BlockSpec((_BR, _M), lambda i: (i, 0)),
            pl.BlockSpec((1, 1), lambda i: (0, 0)),
        ],
        out_shape=[
            jax.ShapeDtypeStruct((_B, _M), jnp.float32),
            jax.ShapeDtypeStruct((1, 1), jnp.float32),
        ],
        scratch_shapes=[
            pltpu.VMEM((1, _M), jnp.float32),
            pltpu.VMEM((1, _M), jnp.float32),
        ],
        compiler_params=pltpu.CompilerParams(
            dimension_semantics=("arbitrary",),
        ),
    )(param, x3, Degraded_feature, w3x, w_gate_text)

    return (gates, loss.reshape(()))
